# Initial kernel scaffold; baseline (speedup 1.0000x reference)
import functools

import jax
import jax.numpy as jnp
from jax import lax
from jax.experimental import pallas as pl
from jax.experimental.pallas import tpu as pltpu
from jax.experimental.pallas import tpu_sc as plsc

N = 10000
E = 320000
D = 128

NC = 2
NS = 16
NW = NC * NS
RPW = 313
NPAD = NW * RPW
DUMMY = RPW

CH = 4000
NCHUNK = E // CH
NV = CH // 16
K = 512
RING = 2 * K

BR = 1250


def _mm_kernel(x_ref, wt_ref, b_ref, o_ref):
    o_ref[...] = (
        jnp.dot(x_ref[...], wt_ref[...], preferred_element_type=jnp.float32)
        + b_ref[...]
    )


def _matmul_bias(x, wt, b):
    return pl.pallas_call(
        _mm_kernel,
        grid=(N // BR,),
        in_specs=[
            pl.BlockSpec((BR, D), lambda i: (i, 0)),
            pl.BlockSpec((D, D), lambda i: (0, 0)),
            pl.BlockSpec((1, D), lambda i: (0, 0)),
        ],
        out_specs=pl.BlockSpec((BR, D), lambda i: (i, 0)),
        out_shape=jax.ShapeDtypeStruct((N, D), jnp.float32),
    )(x, wt, b)


def _out_kernel(x_ref, c_ref, w1t_ref, w2t_ref, b_ref, o_ref):
    c = c_ref[...]
    c = jnp.where(c > -jnp.inf, c, 0.0)
    d = jnp.dot(x_ref[...], w1t_ref[...], preferred_element_type=jnp.float32)
    d = d + jnp.dot(c, w2t_ref[...], preferred_element_type=jnp.float32)
    d = d + b_ref[...]
    nrm = jnp.sqrt(jnp.sum(d * d, axis=1, keepdims=True))
    o_ref[...] = d / jnp.maximum(nrm, 1e-12)


def _combine(x, c_pad, w1t, w2t, b):
    return pl.pallas_call(
        _out_kernel,
        grid=(N // BR,),
        in_specs=[
            pl.BlockSpec((BR, D), lambda i: (i, 0)),
            pl.BlockSpec((BR, D), lambda i: (i, 0)),
            pl.BlockSpec((D, D), lambda i: (0, 0)),
            pl.BlockSpec((D, D), lambda i: (0, 0)),
            pl.BlockSpec((1, D), lambda i: (0, 0)),
        ],
        out_specs=pl.BlockSpec((BR, D), lambda i: (i, 0)),
        out_shape=jax.ShapeDtypeStruct((N, D), jnp.float32),
    )(x, c_pad, w1t, w2t, b)


def _segmax_body(y_hbm, src_hbm, dst_hbm, out_hbm,
                 src_buf, dst_buf, cidx, cdst, rows_v, acc, sem):
    cid = lax.axis_index("c")
    sid = lax.axis_index("s")
    wid = sid * NC + cid
    lo = wid * RPW

    neg = jnp.full((16,), -jnp.inf, dtype=jnp.float32)

    def init_acc(i, carry):
        for ch8 in range(D // 16):
            acc[i, pl.ds(ch8 * 16, 16)] = neg
        return carry

    lax.fori_loop(0, RPW + 1, init_acc, 0)

    zero16 = jnp.zeros((16,), jnp.int32)
    dummy16 = jnp.full((16,), DUMMY, jnp.int32)

    def init_ring(i, carry):
        cidx[pl.ds(i * 16, 16)] = zero16
        cdst[pl.ds(i * 16, 16)] = dummy16
        return carry

    lax.fori_loop(0, RING // 16, init_ring, 0)

    def flush(base):
        pltpu.async_copy(y_hbm.at[cidx.at[pl.ds(base, K)]], rows_v, sem).wait()

        def fold(e, carry):
            d_row = cdst[base + e]
            for ch8 in range(D // 16):
                sl = pl.ds(ch8 * 16, 16)
                acc[d_row, sl] = jnp.maximum(acc[d_row, sl], rows_v[e, sl])
            return carry

        lax.fori_loop(0, K, fold, 0)

    def chunk_body(chunk, nacc):
        pltpu.sync_copy(src_hbm.at[pl.ds(chunk * CH, CH)], src_buf)
        pltpu.sync_copy(dst_hbm.at[pl.ds(chunk * CH, CH)], dst_buf)

        def vec_body(i, nacc):
            dv = dst_buf[pl.ds(i * 16, 16)]
            sv = src_buf[pl.ds(i * 16, 16)]
            m = (dv >= lo) & (dv < lo + RPW)
            mi = m.astype(jnp.int32)
            inc = plsc.cumsum(mi)
            pos = (nacc + inc - 1) & (RING - 1)
            plsc.store_scatter(cidx, [pos], sv, m)
            plsc.store_scatter(cdst, [pos], dv - lo, m)
            nacc2 = nacc + jnp.sum(mi)
            b_old = nacc // K
            crossed = (nacc2 // K) != b_old

            @pl.when(crossed & (b_old % 2 == 0))
            def _():
                flush(0)

            @pl.when(crossed & (b_old % 2 == 1))
            def _():
                flush(K)

            return nacc2

        return lax.fori_loop(0, NV, vec_body, nacc)

    lax.fori_loop(0, NCHUNK, chunk_body, 0)

    flush(0)
    flush(K)
    pltpu.sync_copy(acc.at[pl.ds(0, RPW)], out_hbm.at[pl.ds(lo, RPW)])


_segmax = functools.partial(
    pl.kernel,
    out_type=jax.ShapeDtypeStruct((NPAD, D), jnp.float32),
    mesh=plsc.VectorSubcoreMesh(core_axis_name="c", subcore_axis_name="s"),
    scratch_types=[
        pltpu.VMEM((CH,), jnp.int32),
        pltpu.VMEM((CH,), jnp.int32),
        pltpu.VMEM((RING,), jnp.int32),
        pltpu.VMEM((RING,), jnp.int32),
        pltpu.VMEM((K, D), jnp.float32),
        pltpu.VMEM((RPW + 1, D), jnp.float32),
        pltpu.SemaphoreType.DMA,
    ],
)(_segmax_body)


def kernel(x, edge_index, W_gc, b_gc, W_ma, b_ma, W_ll, b_ll):
    src = edge_index[0]
    dst = edge_index[1]
    y = _matmul_bias(x, W_ma.T, b_ma.reshape(1, D))
    c_pad = _segmax(y, src, dst)
    d = _combine(x, c_pad, W_ll[:, :D].T, W_ll[:, D:].T, b_ll.reshape(1, D))
    return d


# speedup vs baseline: 2.9932x; 2.9932x over previous
import functools

import jax
import jax.numpy as jnp
from jax import lax
from jax.experimental import pallas as pl
from jax.experimental.pallas import tpu as pltpu
from jax.experimental.pallas import tpu_sc as plsc

N = 10000
E = 320000
D = 128

NC = 2
NS = 16
NW = NC * NS
RPW = 320
NPAD = NW * RPW
DUMMY = RPW

CH = 4000
NCHUNK = E // CH
NV = CH // 16
K = 512
RING = 2 * K

BR = 2000


def _mm_kernel(x_ref, wt_ref, b_ref, o_ref):
    o_ref[...] = (
        jnp.dot(x_ref[...], wt_ref[...], preferred_element_type=jnp.float32)
        + b_ref[...]
    )


def _matmul_bias(x, wt, b):
    return pl.pallas_call(
        _mm_kernel,
        grid=(N // BR,),
        in_specs=[
            pl.BlockSpec((BR, D), lambda i: (i, 0)),
            pl.BlockSpec((D, D), lambda i: (0, 0)),
            pl.BlockSpec((1, D), lambda i: (0, 0)),
        ],
        out_specs=pl.BlockSpec((BR, D), lambda i: (i, 0)),
        out_shape=jax.ShapeDtypeStruct((N, D), jnp.float32),
    )(x, wt, b)


def _out_kernel(x_ref, c_ref, w1t_ref, w2t_ref, b_ref, o_ref):
    c = c_ref[...]
    c = jnp.where(c > -jnp.inf, c, 0.0)
    d = jnp.dot(x_ref[...], w1t_ref[...], preferred_element_type=jnp.float32)
    d = d + jnp.dot(c, w2t_ref[...], preferred_element_type=jnp.float32)
    d = d + b_ref[...]
    nrm = jnp.sqrt(jnp.sum(d * d, axis=1, keepdims=True))
    o_ref[...] = d / jnp.maximum(nrm, 1e-12)


def _combine(x, c_pad, w1t, w2t, b):
    return pl.pallas_call(
        _out_kernel,
        grid=(N // BR,),
        in_specs=[
            pl.BlockSpec((BR, D), lambda i: (i, 0)),
            pl.BlockSpec((BR, D), lambda i: (i, 0)),
            pl.BlockSpec((D, D), lambda i: (0, 0)),
            pl.BlockSpec((D, D), lambda i: (0, 0)),
            pl.BlockSpec((1, D), lambda i: (0, 0)),
        ],
        out_specs=pl.BlockSpec((BR, D), lambda i: (i, 0)),
        out_shape=jax.ShapeDtypeStruct((N, D), jnp.float32),
    )(x, c_pad, w1t, w2t, b)


def _segmax_body(y_hbm, src_hbm, dst_hbm, out_hbm,
                 src_buf, dst_buf, cidx, cdst, rows_v, acc, sem):
    cid = lax.axis_index("c")
    sid = lax.axis_index("s")
    wid = sid * NC + cid
    lo = wid * RPW

    neg = jnp.full((16,), -jnp.inf, dtype=jnp.float32)

    def init_acc(i, carry):
        for ch8 in range(D // 16):
            acc[i, pl.ds(ch8 * 16, 16)] = neg
        return carry

    lax.fori_loop(0, RPW + 1, init_acc, 0)

    zero16 = jnp.zeros((16,), jnp.int32)
    dummy16 = jnp.full((16,), DUMMY, jnp.int32)

    def init_ring(i, carry):
        cidx[pl.ds(i * 16, 16)] = zero16
        cdst[pl.ds(i * 16, 16)] = dummy16
        return carry

    lax.fori_loop(0, (K + 16) // 16, init_ring, 0)

    def flush():
        pltpu.async_copy(y_hbm.at[cidx.at[pl.ds(0, K)]], rows_v, sem).wait()

        def fold16(g, carry):
            dvec = cdst[pl.ds(g * 16, 16)]
            for lane in range(16):
                d_row = dvec[lane]
                e = g * 16 + lane
                for ch8 in range(D // 16):
                    sl = pl.ds(ch8 * 16, 16)
                    acc[d_row, sl] = jnp.maximum(acc[d_row, sl], rows_v[e, sl])
            return carry

        lax.fori_loop(0, K // 16, fold16, 0)

    def chunk_body(chunk, nacc):
        pltpu.sync_copy(src_hbm.at[pl.ds(chunk * CH, CH)], src_buf)
        pltpu.sync_copy(dst_hbm.at[pl.ds(chunk * CH, CH)], dst_buf)

        def vec_body(i, nacc):
            dv = dst_buf[pl.ds(i * 16, 16)]
            sv = src_buf[pl.ds(i * 16, 16)]
            m = (dv >= lo) & (dv < lo + RPW)
            plsc.store_compressed(cidx.at[pl.ds(nacc, 16)], sv, mask=m)
            plsc.store_compressed(cdst.at[pl.ds(nacc, 16)], dv - lo, mask=m)
            cnt = plsc.all_reduce_population_count(m)[0]
            nacc2 = nacc + cnt

            @pl.when(nacc2 >= K)
            def _():
                flush()
                ov_i = cidx[pl.ds(K, 16)]
                ov_d = cdst[pl.ds(K, 16)]
                cidx[pl.ds(0, 16)] = ov_i
                cdst[pl.ds(0, 16)] = ov_d

            return jnp.where(nacc2 >= K, nacc2 - K, nacc2)

        return lax.fori_loop(0, NV, vec_body, nacc)

    lax.fori_loop(0, NCHUNK, chunk_body, 0)

    flush()
    pltpu.sync_copy(acc.at[pl.ds(0, RPW)], out_hbm.at[pl.ds(lo, RPW)])


_segmax = functools.partial(
    pl.kernel,
    out_type=jax.ShapeDtypeStruct((NPAD, D), jnp.float32),
    mesh=plsc.VectorSubcoreMesh(core_axis_name="c", subcore_axis_name="s"),
    compiler_params=pltpu.CompilerParams(needs_layout_passes=False),
    scratch_types=[
        pltpu.VMEM((CH,), jnp.int32),
        pltpu.VMEM((CH,), jnp.int32),
        pltpu.VMEM((K + 16,), jnp.int32),
        pltpu.VMEM((K + 16,), jnp.int32),
        pltpu.VMEM((K, D), jnp.float32),
        pltpu.VMEM((RPW + 1, D), jnp.float32),
        pltpu.SemaphoreType.DMA,
    ],
)(_segmax_body)


def kernel(x, edge_index, W_gc, b_gc, W_ma, b_ma, W_ll, b_ll):
    src = edge_index[0]
    dst = edge_index[1]
    y = _matmul_bias(x, W_ma.T, b_ma.reshape(1, D))
    c_pad = _segmax(y, src, dst)
    d = _combine(x, c_pad, W_ll[:, :D].T, W_ll[:, D:].T, b_ll.reshape(1, D))
    return d


# speedup vs baseline: 5.8535x; 1.9556x over previous
import functools

import jax
import jax.numpy as jnp
from jax import lax
from jax.experimental import pallas as pl
from jax.experimental.pallas import tpu as pltpu
from jax.experimental.pallas import tpu_sc as plsc

N = 10000
E = 320000
D = 128

NC = 2
NS = 16
NW = NC * NS
RPW = 320
NPAD = NW * RPW
DUMMY = RPW

CH = 3200
NCHUNK = E // CH
NV = CH // 16
K = 512
PAD = 64
UNROLL = 4

BR = 2000


def _mm_kernel(x_ref, wt_ref, b_ref, o_ref):
    o_ref[...] = (
        jnp.dot(x_ref[...], wt_ref[...], preferred_element_type=jnp.float32)
        + b_ref[...]
    )


def _matmul_bias(x, wt, b):
    return pl.pallas_call(
        _mm_kernel,
        grid=(N // BR,),
        in_specs=[
            pl.BlockSpec((BR, D), lambda i: (i, 0)),
            pl.BlockSpec((D, D), lambda i: (0, 0)),
            pl.BlockSpec((1, D), lambda i: (0, 0)),
        ],
        out_specs=pl.BlockSpec((BR, D), lambda i: (i, 0)),
        out_shape=jax.ShapeDtypeStruct((N, D), jnp.float32),
    )(x, wt, b)


def _out_kernel(x_ref, c_ref, w1t_ref, w2t_ref, b_ref, o_ref):
    c = c_ref[...]
    c = jnp.where(c > -jnp.inf, c, 0.0)
    d = jnp.dot(x_ref[...], w1t_ref[...], preferred_element_type=jnp.float32)
    d = d + jnp.dot(c, w2t_ref[...], preferred_element_type=jnp.float32)
    d = d + b_ref[...]
    nrm = jnp.sqrt(jnp.sum(d * d, axis=1, keepdims=True))
    o_ref[...] = d / jnp.maximum(nrm, 1e-12)


def _combine(x, c_pad, w1t, w2t, b):
    return pl.pallas_call(
        _out_kernel,
        grid=(N // BR,),
        in_specs=[
            pl.BlockSpec((BR, D), lambda i: (i, 0)),
            pl.BlockSpec((BR, D), lambda i: (i, 0)),
            pl.BlockSpec((D, D), lambda i: (0, 0)),
            pl.BlockSpec((D, D), lambda i: (0, 0)),
            pl.BlockSpec((1, D), lambda i: (0, 0)),
        ],
        out_specs=pl.BlockSpec((BR, D), lambda i: (i, 0)),
        out_shape=jax.ShapeDtypeStruct((N, D), jnp.float32),
    )(x, c_pad, w1t, w2t, b)


def _segmax_body(y_hbm, src_hbm, dst_hbm, out_hbm,
                 src_buf, dst_buf, cidx, cdst, rows_v, acc, sem):
    cid = lax.axis_index("c")
    sid = lax.axis_index("s")
    wid = sid * NC + cid
    lo = wid * RPW

    neg = jnp.full((16,), -jnp.inf, dtype=jnp.float32)

    def init_acc(i, carry):
        for ch8 in range(D // 16):
            acc[i, pl.ds(ch8 * 16, 16)] = neg
        return carry

    lax.fori_loop(0, RPW + 1, init_acc, 0)

    zero16 = jnp.zeros((16,), jnp.int32)
    dummy16 = jnp.full((16,), DUMMY, jnp.int32)

    def init_ring(i, carry):
        cidx[pl.ds(i * 16, 16)] = zero16
        cdst[pl.ds(i * 16, 16)] = dummy16
        return carry

    lax.fori_loop(0, (K + PAD) // 16, init_ring, 0)

    def flush():
        pltpu.async_copy(y_hbm.at[cidx.at[pl.ds(0, K)]], rows_v, sem).wait()

        def fold16(g, carry):
            dvec = cdst[pl.ds(g * 16, 16)]
            for lane in range(16):
                d_row = dvec[lane]
                e = g * 16 + lane
                row = [rows_v[e, pl.ds(c * 16, 16)] for c in range(D // 16)]
                cur = [acc[d_row, pl.ds(c * 16, 16)] for c in range(D // 16)]
                for c in range(D // 16):
                    acc[d_row, pl.ds(c * 16, 16)] = jnp.maximum(cur[c], row[c])
            return carry

        lax.fori_loop(0, K // 16, fold16, 0)

    def chunk_body(chunk, nacc):
        pltpu.sync_copy(src_hbm.at[pl.ds(chunk * CH, CH)], src_buf)
        pltpu.sync_copy(dst_hbm.at[pl.ds(chunk * CH, CH)], dst_buf)

        def grp_body(g, nacc):
            svs, dls, ms, cnts = [], [], [], []
            for u in range(UNROLL):
                i = g * UNROLL + u
                dv = dst_buf[pl.ds(i * 16, 16)]
                sv = src_buf[pl.ds(i * 16, 16)]
                dl = dv - lo
                m = dl.astype(jnp.uint32) < jnp.uint32(RPW)
                cnts.append(plsc.all_reduce_population_count(m)[0])
                svs.append(sv)
                dls.append(dl)
                ms.append(m)
            n = nacc
            for u in range(UNROLL):
                plsc.store_compressed(cidx.at[pl.ds(n, 16)], svs[u], mask=ms[u])
                plsc.store_compressed(cdst.at[pl.ds(n, 16)], dls[u], mask=ms[u])
                n = n + cnts[u]

            @pl.when(n >= K)
            def _():
                flush()
                for t in range(PAD // 16):
                    ov_i = cidx[pl.ds(K + t * 16, 16)]
                    ov_d = cdst[pl.ds(K + t * 16, 16)]
                    cidx[pl.ds(t * 16, 16)] = ov_i
                    cdst[pl.ds(t * 16, 16)] = ov_d

            return jnp.where(n >= K, n - K, n)

        return lax.fori_loop(0, NV // UNROLL, grp_body, nacc)

    lax.fori_loop(0, NCHUNK, chunk_body, 0)

    flush()
    pltpu.sync_copy(acc.at[pl.ds(0, RPW)], out_hbm.at[pl.ds(lo, RPW)])


_segmax = functools.partial(
    pl.kernel,
    out_type=jax.ShapeDtypeStruct((NPAD, D), jnp.float32),
    mesh=plsc.VectorSubcoreMesh(core_axis_name="c", subcore_axis_name="s"),
    compiler_params=pltpu.CompilerParams(needs_layout_passes=False),
    scratch_types=[
        pltpu.VMEM((CH,), jnp.int32),
        pltpu.VMEM((CH,), jnp.int32),
        pltpu.VMEM((K + PAD,), jnp.int32),
        pltpu.VMEM((K + PAD,), jnp.int32),
        pltpu.VMEM((K, D), jnp.float32),
        pltpu.VMEM((RPW + 1, D), jnp.float32),
        pltpu.SemaphoreType.DMA,
    ],
)(_segmax_body)


def kernel(x, edge_index, W_gc, b_gc, W_ma, b_ma, W_ll, b_ll):
    src = edge_index[0]
    dst = edge_index[1]
    y = _matmul_bias(x, W_ma.T, b_ma.reshape(1, D))
    c_pad = _segmax(y, src, dst)
    d = _combine(x, c_pad, W_ll[:, :D].T, W_ll[:, D:].T, b_ll.reshape(1, D))
    return d


# speedup vs baseline: 7.3158x; 1.2498x over previous
import functools

import jax
import jax.numpy as jnp
from jax import lax
from jax.experimental import pallas as pl
from jax.experimental.pallas import tpu as pltpu
from jax.experimental.pallas import tpu_sc as plsc

N = 10000
E = 320000
D = 128

NC = 2
NS = 16
NW = NC * NS
RPW = 320
NPAD = NW * RPW
DUMMY = RPW

CH = 3200
NCHUNK = E // CH
NV = CH // 16
K = 512
PAD = 64
UNROLL = 4

BR = 2000


def _mm_kernel(x_ref, wt_ref, b_ref, o_ref):
    o_ref[...] = (
        jnp.dot(x_ref[...], wt_ref[...], preferred_element_type=jnp.float32)
        + b_ref[...]
    )


def _matmul_bias(x, wt, b):
    return pl.pallas_call(
        _mm_kernel,
        grid=(N // BR,),
        in_specs=[
            pl.BlockSpec((BR, D), lambda i: (i, 0)),
            pl.BlockSpec((D, D), lambda i: (0, 0)),
            pl.BlockSpec((1, D), lambda i: (0, 0)),
        ],
        out_specs=pl.BlockSpec((BR, D), lambda i: (i, 0)),
        out_shape=jax.ShapeDtypeStruct((N, D), jnp.float32),
    )(x, wt, b)


def _out_kernel(x_ref, c_ref, w1t_ref, w2t_ref, b_ref, o_ref):
    c = c_ref[...]
    c = jnp.where(c > -jnp.inf, c, 0.0)
    d = jnp.dot(x_ref[...], w1t_ref[...], preferred_element_type=jnp.float32)
    d = d + jnp.dot(c, w2t_ref[...], preferred_element_type=jnp.float32)
    d = d + b_ref[...]
    nrm = jnp.sqrt(jnp.sum(d * d, axis=1, keepdims=True))
    o_ref[...] = d / jnp.maximum(nrm, 1e-12)


def _combine(x, c_pad, w1t, w2t, b):
    return pl.pallas_call(
        _out_kernel,
        grid=(N // BR,),
        in_specs=[
            pl.BlockSpec((BR, D), lambda i: (i, 0)),
            pl.BlockSpec((BR, D), lambda i: (i, 0)),
            pl.BlockSpec((D, D), lambda i: (0, 0)),
            pl.BlockSpec((D, D), lambda i: (0, 0)),
            pl.BlockSpec((1, D), lambda i: (0, 0)),
        ],
        out_specs=pl.BlockSpec((BR, D), lambda i: (i, 0)),
        out_shape=jax.ShapeDtypeStruct((N, D), jnp.float32),
    )(x, c_pad, w1t, w2t, b)


def _segmax_body(y_hbm, src_hbm, dst_hbm, out_hbm,
                 src_buf, dst_buf, cidx, cdst, rows_v, acc, sem, sem_s, sem_d):
    cid = lax.axis_index("c")
    sid = lax.axis_index("s")
    wid = sid * NC + cid
    lo = wid * RPW

    neg = jnp.full((16,), -jnp.inf, dtype=jnp.float32)

    def init_acc(i, carry):
        for ch8 in range(D // 16):
            acc[i, pl.ds(ch8 * 16, 16)] = neg
        return carry

    lax.fori_loop(0, RPW + 1, init_acc, 0)

    zero16 = jnp.zeros((16,), jnp.int32)
    dummy16 = jnp.full((16,), DUMMY, jnp.int32)

    def init_ring(i, carry):
        cidx[pl.ds(i * 16, 16)] = zero16
        cdst[pl.ds(i * 16, 16)] = dummy16
        return carry

    lax.fori_loop(0, (K + PAD) // 16, init_ring, 0)

    def flush():
        pltpu.async_copy(y_hbm.at[cidx.at[pl.ds(0, K)]], rows_v, sem).wait()

        def fold16(g, carry):
            dvec = cdst[pl.ds(g * 16, 16)]
            for lane in range(16):
                d_row = dvec[lane]
                e = g * 16 + lane
                row = [rows_v[e, pl.ds(c * 16, 16)] for c in range(D // 16)]
                cur = [acc[d_row, pl.ds(c * 16, 16)] for c in range(D // 16)]
                for c in range(D // 16):
                    acc[d_row, pl.ds(c * 16, 16)] = jnp.maximum(cur[c], row[c])
            return carry

        lax.fori_loop(0, K // 16, fold16, 0)

    def start_fetch(c, par):
        pltpu.async_copy(src_hbm.at[pl.ds(c * CH, CH)], src_buf.at[par], sem_s)
        pltpu.async_copy(dst_hbm.at[pl.ds(c * CH, CH)], dst_buf.at[par], sem_d)

    start_fetch(0, 0)

    def chunk_body(chunk, nacc):
        par = jnp.bitwise_and(chunk, 1)
        pltpu.make_async_copy(src_hbm.at[pl.ds(chunk * CH, CH)],
                              src_buf.at[par], sem_s).wait()
        pltpu.make_async_copy(dst_hbm.at[pl.ds(chunk * CH, CH)],
                              dst_buf.at[par], sem_d).wait()

        @pl.when(chunk + 1 < NCHUNK)
        def _():
            start_fetch(chunk + 1, 1 - par)

        def grp_body(g, nacc):
            svs, dls, ms, cnts = [], [], [], []
            for u in range(UNROLL):
                i = g * UNROLL + u
                dv = dst_buf[par, pl.ds(i * 16, 16)]
                sv = src_buf[par, pl.ds(i * 16, 16)]
                dl = dv - lo
                m = dl.astype(jnp.uint32) < jnp.uint32(RPW)
                cnts.append(plsc.all_reduce_population_count(m)[0])
                svs.append(sv)
                dls.append(dl)
                ms.append(m)
            n = nacc
            for u in range(UNROLL):
                plsc.store_compressed(cidx.at[pl.ds(n, 16)], svs[u], mask=ms[u])
                plsc.store_compressed(cdst.at[pl.ds(n, 16)], dls[u], mask=ms[u])
                n = n + cnts[u]

            @pl.when(n >= K)
            def _():
                flush()
                for t in range(PAD // 16):
                    ov_i = cidx[pl.ds(K + t * 16, 16)]
                    ov_d = cdst[pl.ds(K + t * 16, 16)]
                    cidx[pl.ds(t * 16, 16)] = ov_i
                    cdst[pl.ds(t * 16, 16)] = ov_d

            return jnp.where(n >= K, n - K, n)

        return lax.fori_loop(0, NV // UNROLL, grp_body, nacc)

    lax.fori_loop(0, NCHUNK, chunk_body, 0)

    flush()
    pltpu.sync_copy(acc.at[pl.ds(0, RPW)], out_hbm.at[pl.ds(lo, RPW)])


_segmax = functools.partial(
    pl.kernel,
    out_type=jax.ShapeDtypeStruct((NPAD, D), jnp.float32),
    mesh=plsc.VectorSubcoreMesh(core_axis_name="c", subcore_axis_name="s"),
    compiler_params=pltpu.CompilerParams(needs_layout_passes=False),
    scratch_types=[
        pltpu.VMEM((2, CH), jnp.int32),
        pltpu.VMEM((2, CH), jnp.int32),
        pltpu.VMEM((K + PAD,), jnp.int32),
        pltpu.VMEM((K + PAD,), jnp.int32),
        pltpu.VMEM((K, D), jnp.float32),
        pltpu.VMEM((RPW + 1, D), jnp.float32),
        pltpu.SemaphoreType.DMA,
        pltpu.SemaphoreType.DMA,
        pltpu.SemaphoreType.DMA,
    ],
)(_segmax_body)


def kernel(x, edge_index, W_gc, b_gc, W_ma, b_ma, W_ll, b_ll):
    src = edge_index[0]
    dst = edge_index[1]
    y = _matmul_bias(x, W_ma.T, b_ma.reshape(1, D))
    c_pad = _segmax(y, src, dst)
    d = _combine(x, c_pad, W_ll[:, :D].T, W_ll[:, D:].T, b_ll.reshape(1, D))
    return d


# speedup vs baseline: 8.4185x; 1.1507x over previous
import functools

import jax
import jax.numpy as jnp
from jax import lax
from jax.experimental import pallas as pl
from jax.experimental.pallas import tpu as pltpu
from jax.experimental.pallas import tpu_sc as plsc

N = 10000
E = 320000
D = 128

NC = 2
NS = 16
NW = NC * NS
RPW = 320
NPAD = NW * RPW
DUMMY = RPW

CH = 3200
NCHUNK = E // CH
NV = CH // 16
K = 256
PAD = 64
UNROLL = 4

BR = 2000


def _mm_kernel(x_ref, wt_ref, b_ref, o_ref):
    o_ref[...] = (
        jnp.dot(x_ref[...], wt_ref[...], preferred_element_type=jnp.float32)
        + b_ref[...]
    )


def _matmul_bias(x, wt, b):
    return pl.pallas_call(
        _mm_kernel,
        grid=(N // BR,),
        in_specs=[
            pl.BlockSpec((BR, D), lambda i: (i, 0)),
            pl.BlockSpec((D, D), lambda i: (0, 0)),
            pl.BlockSpec((1, D), lambda i: (0, 0)),
        ],
        out_specs=pl.BlockSpec((BR, D), lambda i: (i, 0)),
        out_shape=jax.ShapeDtypeStruct((N, D), jnp.float32),
    )(x, wt, b)


def _out_kernel(x_ref, c_ref, w1t_ref, w2t_ref, b_ref, o_ref):
    c = c_ref[...]
    c = jnp.where(c > -jnp.inf, c, 0.0)
    d = jnp.dot(x_ref[...], w1t_ref[...], preferred_element_type=jnp.float32)
    d = d + jnp.dot(c, w2t_ref[...], preferred_element_type=jnp.float32)
    d = d + b_ref[...]
    nrm = jnp.sqrt(jnp.sum(d * d, axis=1, keepdims=True))
    o_ref[...] = d / jnp.maximum(nrm, 1e-12)


def _combine(x, c_pad, w1t, w2t, b):
    return pl.pallas_call(
        _out_kernel,
        grid=(N // BR,),
        in_specs=[
            pl.BlockSpec((BR, D), lambda i: (i, 0)),
            pl.BlockSpec((BR, D), lambda i: (i, 0)),
            pl.BlockSpec((D, D), lambda i: (0, 0)),
            pl.BlockSpec((D, D), lambda i: (0, 0)),
            pl.BlockSpec((1, D), lambda i: (0, 0)),
        ],
        out_specs=pl.BlockSpec((BR, D), lambda i: (i, 0)),
        out_shape=jax.ShapeDtypeStruct((N, D), jnp.float32),
    )(x, c_pad, w1t, w2t, b)


def _segmax_body(y_hbm, src_hbm, dst_hbm, out_hbm,
                 src_buf, dst_buf, cidx, cdst, gidx, rows_v, acc,
                 sem, sem_s, sem_d):
    cid = lax.axis_index("c")
    sid = lax.axis_index("s")
    wid = sid * NC + cid
    lo = wid * RPW

    neg = jnp.full((16,), -jnp.inf, dtype=jnp.float32)

    def init_acc(i, carry):
        for ch8 in range(D // 16):
            acc[i, pl.ds(ch8 * 16, 16)] = neg
        return carry

    lax.fori_loop(0, RPW + 1, init_acc, 0)

    zero16 = jnp.zeros((16,), jnp.int32)
    dummy16 = jnp.full((16,), DUMMY, jnp.int32)

    def init_ring(i, carry):
        for p in range(2):
            cidx[p, pl.ds(i * 16, 16)] = zero16
            cdst[p, pl.ds(i * 16, 16)] = dummy16
        return carry

    lax.fori_loop(0, (K + PAD) // 16, init_ring, 0)

    def fire(p):
        for t in range(K // 16):
            gidx[pl.ds(t * 16, 16)] = cidx[p, pl.ds(t * 16, 16)]
        pltpu.async_copy(y_hbm.at[gidx], rows_v.at[p], sem)

    def wait_fold(p):
        pltpu.make_async_copy(y_hbm.at[gidx], rows_v.at[p], sem).wait()

        def fold16(g, carry):
            dvec = cdst[p, pl.ds(g * 16, 16)]
            for lane in range(16):
                d_row = dvec[lane]
                e = g * 16 + lane
                row = [rows_v[p, e, pl.ds(c * 16, 16)] for c in range(D // 16)]
                cur = [acc[d_row, pl.ds(c * 16, 16)] for c in range(D // 16)]
                for c in range(D // 16):
                    acc[d_row, pl.ds(c * 16, 16)] = jnp.maximum(cur[c], row[c])
            return carry

        lax.fori_loop(0, K // 16, fold16, 0)

    def start_fetch(c, par):
        pltpu.async_copy(src_hbm.at[pl.ds(c * CH, CH)], src_buf.at[par], sem_s)
        pltpu.async_copy(dst_hbm.at[pl.ds(c * CH, CH)], dst_buf.at[par], sem_d)

    start_fetch(0, 0)

    def chunk_body(chunk, carry):
        nacc = carry
        par = jnp.bitwise_and(chunk, 1)
        pltpu.make_async_copy(src_hbm.at[pl.ds(chunk * CH, CH)],
                              src_buf.at[par], sem_s).wait()
        pltpu.make_async_copy(dst_hbm.at[pl.ds(chunk * CH, CH)],
                              dst_buf.at[par], sem_d).wait()

        @pl.when(chunk + 1 < NCHUNK)
        def _():
            start_fetch(chunk + 1, 1 - par)

        def grp_body(g, carry):
            nacc, pend, bp = carry
            svs, dls, ms, cnts = [], [], [], []
            for u in range(UNROLL):
                i = g * UNROLL + u
                dv = dst_buf[par, pl.ds(i * 16, 16)]
                sv = src_buf[par, pl.ds(i * 16, 16)]
                dl = dv - lo
                m = dl.astype(jnp.uint32) < jnp.uint32(RPW)
                cnts.append(plsc.all_reduce_population_count(m)[0])
                svs.append(sv)
                dls.append(dl)
                ms.append(m)
            n = nacc
            for u in range(UNROLL):
                plsc.store_compressed(cidx.at[bp, pl.ds(n, 16)], svs[u],
                                      mask=ms[u])
                plsc.store_compressed(cdst.at[bp, pl.ds(n, 16)], dls[u],
                                      mask=ms[u])
                n = n + cnts[u]
            full = n >= K

            @pl.when(full)
            def _():
                @pl.when(pend == 1)
                def _():
                    wait_fold(1 - bp)

                fire(bp)
                for t in range(PAD // 16):
                    ov_i = cidx[bp, pl.ds(K + t * 16, 16)]
                    ov_d = cdst[bp, pl.ds(K + t * 16, 16)]
                    cidx[1 - bp, pl.ds(t * 16, 16)] = ov_i
                    cdst[1 - bp, pl.ds(t * 16, 16)] = ov_d

            nacc2 = jnp.where(full, n - K, n)
            pend2 = jnp.where(full, 1, pend)
            bp2 = jnp.where(full, 1 - bp, bp)
            return (nacc2, pend2, bp2)

        return lax.fori_loop(0, NV // UNROLL, grp_body, nacc)

    nacc, pend, bp = lax.fori_loop(0, NCHUNK, chunk_body, (0, 0, 0))

    @pl.when(pend == 1)
    def _():
        wait_fold(1 - bp)

    fire(bp)
    wait_fold(bp)
    pltpu.sync_copy(acc.at[pl.ds(0, RPW)], out_hbm.at[pl.ds(lo, RPW)])


_segmax = functools.partial(
    pl.kernel,
    out_type=jax.ShapeDtypeStruct((NPAD, D), jnp.float32),
    mesh=plsc.VectorSubcoreMesh(core_axis_name="c", subcore_axis_name="s"),
    compiler_params=pltpu.CompilerParams(needs_layout_passes=False),
    scratch_types=[
        pltpu.VMEM((2, CH), jnp.int32),
        pltpu.VMEM((2, CH), jnp.int32),
        pltpu.VMEM((2, K + PAD), jnp.int32),
        pltpu.VMEM((2, K + PAD), jnp.int32),
        pltpu.VMEM((K,), jnp.int32),
        pltpu.VMEM((2, K, D), jnp.float32),
        pltpu.VMEM((RPW + 1, D), jnp.float32),
        pltpu.SemaphoreType.DMA,
        pltpu.SemaphoreType.DMA,
        pltpu.SemaphoreType.DMA,
    ],
)(_segmax_body)


def kernel(x, edge_index, W_gc, b_gc, W_ma, b_ma, W_ll, b_ll):
    src = edge_index[0]
    dst = edge_index[1]
    y = _matmul_bias(x, W_ma.T, b_ma.reshape(1, D))
    c_pad = _segmax(y, src, dst)
    d = _combine(x, c_pad, W_ll[:, :D].T, W_ll[:, D:].T, b_ll.reshape(1, D))
    return d


# speedup vs baseline: 8.6580x; 1.0284x over previous
import functools

import jax
import jax.numpy as jnp
from jax import lax
from jax.experimental import pallas as pl
from jax.experimental.pallas import tpu as pltpu
from jax.experimental.pallas import tpu_sc as plsc

N = 10000
E = 320000
D = 128

NC = 2
NS = 16
NW = NC * NS
RPW = 320
NPAD = NW * RPW
DUMMY = RPW

CH = 3200
NCHUNK = E // CH
NV = CH // 16
K = 256
PAD = 64
UNROLL = 4

BR = 2000


def _mm_kernel(x_ref, wt_ref, b_ref, o_ref):
    o_ref[...] = (
        jnp.dot(x_ref[...], wt_ref[...], preferred_element_type=jnp.float32)
        + b_ref[...]
    )


def _matmul_bias(x, wt, b):
    return pl.pallas_call(
        _mm_kernel,
        grid=(N // BR,),
        in_specs=[
            pl.BlockSpec((BR, D), lambda i: (i, 0)),
            pl.BlockSpec((D, D), lambda i: (0, 0)),
            pl.BlockSpec((1, D), lambda i: (0, 0)),
        ],
        out_specs=pl.BlockSpec((BR, D), lambda i: (i, 0)),
        out_shape=jax.ShapeDtypeStruct((N, D), jnp.float32),
    )(x, wt, b)


def _out_kernel(x_ref, c_ref, w1t_ref, w2t_ref, b_ref, o_ref):
    c = c_ref[...]
    c = jnp.where(c > -jnp.inf, c, 0.0)
    d = jnp.dot(x_ref[...], w1t_ref[...], preferred_element_type=jnp.float32)
    d = d + jnp.dot(c, w2t_ref[...], preferred_element_type=jnp.float32)
    d = d + b_ref[...]
    nrm = jnp.sqrt(jnp.sum(d * d, axis=1, keepdims=True))
    o_ref[...] = d / jnp.maximum(nrm, 1e-12)


def _combine(x, c_pad, w1t, w2t, b):
    return pl.pallas_call(
        _out_kernel,
        grid=(N // BR,),
        in_specs=[
            pl.BlockSpec((BR, D), lambda i: (i, 0)),
            pl.BlockSpec((BR, D), lambda i: (i, 0)),
            pl.BlockSpec((D, D), lambda i: (0, 0)),
            pl.BlockSpec((D, D), lambda i: (0, 0)),
            pl.BlockSpec((1, D), lambda i: (0, 0)),
        ],
        out_specs=pl.BlockSpec((BR, D), lambda i: (i, 0)),
        out_shape=jax.ShapeDtypeStruct((N, D), jnp.float32),
    )(x, c_pad, w1t, w2t, b)


def _segmax_body(y_hbm, src_hbm, dst_hbm, out_hbm,
                 src_buf, dst_buf, cidx0, cdst0, cidx1, cdst1,
                 rows0, rows1, acc, sem, sem_s, sem_d):
    cid = lax.axis_index("c")
    sid = lax.axis_index("s")
    wid = sid * NC + cid
    lo = wid * RPW

    neg = jnp.full((16,), -jnp.inf, dtype=jnp.float32)

    def init_acc(i, carry):
        for ch8 in range(D // 16):
            acc[i, pl.ds(ch8 * 16, 16)] = neg
        return carry

    lax.fori_loop(0, RPW + 1, init_acc, 0)

    zero16 = jnp.zeros((16,), jnp.int32)
    dummy16 = jnp.full((16,), DUMMY, jnp.int32)

    def init_ring(i, carry):
        cidx0[pl.ds(i * 16, 16)] = zero16
        cdst0[pl.ds(i * 16, 16)] = dummy16
        cidx1[pl.ds(i * 16, 16)] = zero16
        cdst1[pl.ds(i * 16, 16)] = dummy16
        return carry

    lax.fori_loop(0, (K + PAD) // 16, init_ring, 0)

    def fire(ci, rv):
        pltpu.async_copy(y_hbm.at[ci.at[pl.ds(0, K)]], rv, sem)

    def wait_fold(ci, cd, rv):
        pltpu.make_async_copy(y_hbm.at[ci.at[pl.ds(0, K)]], rv, sem).wait()

        def fold16(g, carry):
            dvec = cd[pl.ds(g * 16, 16)]
            for lane in range(16):
                d_row = dvec[lane]
                e = g * 16 + lane
                row = [rv[e, pl.ds(c * 16, 16)] for c in range(D // 16)]
                cur = [acc[d_row, pl.ds(c * 16, 16)] for c in range(D // 16)]
                for c in range(D // 16):
                    acc[d_row, pl.ds(c * 16, 16)] = jnp.maximum(cur[c], row[c])
            return carry

        lax.fori_loop(0, K // 16, fold16, 0)

    def ov_copy(src_ci, src_cd, dst_ci, dst_cd):
        for t in range(PAD // 16):
            ov_i = src_ci[pl.ds(K + t * 16, 16)]
            ov_d = src_cd[pl.ds(K + t * 16, 16)]
            dst_ci[pl.ds(t * 16, 16)] = ov_i
            dst_cd[pl.ds(t * 16, 16)] = ov_d

    def start_fetch(c, par):
        pltpu.async_copy(src_hbm.at[pl.ds(c * CH, CH)], src_buf.at[par], sem_s)
        pltpu.async_copy(dst_hbm.at[pl.ds(c * CH, CH)], dst_buf.at[par], sem_d)

    start_fetch(0, 0)

    def chunk_body(chunk, carry):
        par = jnp.bitwise_and(chunk, 1)
        pltpu.make_async_copy(src_hbm.at[pl.ds(chunk * CH, CH)],
                              src_buf.at[par], sem_s).wait()
        pltpu.make_async_copy(dst_hbm.at[pl.ds(chunk * CH, CH)],
                              dst_buf.at[par], sem_d).wait()

        @pl.when(chunk + 1 < NCHUNK)
        def _():
            start_fetch(chunk + 1, 1 - par)

        def grp_body(g, carry):
            nacc, pend, bp = carry
            svs, dls, ms, cnts = [], [], [], []
            for u in range(UNROLL):
                i = g * UNROLL + u
                dv = dst_buf[par, pl.ds(i * 16, 16)]
                sv = src_buf[par, pl.ds(i * 16, 16)]
                dl = dv - lo
                m = dl.astype(jnp.uint32) < jnp.uint32(RPW)
                cnts.append(plsc.all_reduce_population_count(m)[0])
                svs.append(sv)
                dls.append(dl)
                ms.append(m)

            def append(ci, cd):
                n = nacc
                for u in range(UNROLL):
                    plsc.store_compressed(ci.at[pl.ds(n, 16)], svs[u],
                                          mask=ms[u])
                    plsc.store_compressed(cd.at[pl.ds(n, 16)], dls[u],
                                          mask=ms[u])
                    n = n + cnts[u]
                return n

            @pl.when(bp == 0)
            def _():
                append(cidx0, cdst0)

            @pl.when(bp == 1)
            def _():
                append(cidx1, cdst1)

            n = nacc + cnts[0] + cnts[1] + cnts[2] + cnts[3]
            full = n >= K

            @pl.when(full)
            def _():
                @pl.when((pend == 1) & (bp == 0))
                def _():
                    wait_fold(cidx1, cdst1, rows1)

                @pl.when((pend == 1) & (bp == 1))
                def _():
                    wait_fold(cidx0, cdst0, rows0)

                @pl.when(bp == 0)
                def _():
                    fire(cidx0, rows0)
                    ov_copy(cidx0, cdst0, cidx1, cdst1)

                @pl.when(bp == 1)
                def _():
                    fire(cidx1, rows1)
                    ov_copy(cidx1, cdst1, cidx0, cdst0)

            nacc2 = jnp.where(full, n - K, n)
            pend2 = jnp.where(full, 1, pend)
            bp2 = jnp.where(full, 1 - bp, bp)
            return (nacc2, pend2, bp2)

        return lax.fori_loop(0, NV // UNROLL, grp_body, carry)

    nacc, pend, bp = lax.fori_loop(0, NCHUNK, chunk_body, (0, 0, 0))

    @pl.when((pend == 1) & (bp == 0))
    def _():
        wait_fold(cidx1, cdst1, rows1)

    @pl.when((pend == 1) & (bp == 1))
    def _():
        wait_fold(cidx0, cdst0, rows0)

    @pl.when(bp == 0)
    def _():
        fire(cidx0, rows0)
        wait_fold(cidx0, cdst0, rows0)

    @pl.when(bp == 1)
    def _():
        fire(cidx1, rows1)
        wait_fold(cidx1, cdst1, rows1)

    pltpu.sync_copy(acc.at[pl.ds(0, RPW)], out_hbm.at[pl.ds(lo, RPW)])


_segmax = functools.partial(
    pl.kernel,
    out_type=jax.ShapeDtypeStruct((NPAD, D), jnp.float32),
    mesh=plsc.VectorSubcoreMesh(core_axis_name="c", subcore_axis_name="s"),
    compiler_params=pltpu.CompilerParams(needs_layout_passes=False),
    scratch_types=[
        pltpu.VMEM((2, CH), jnp.int32),
        pltpu.VMEM((2, CH), jnp.int32),
        pltpu.VMEM((K + PAD,), jnp.int32),
        pltpu.VMEM((K + PAD,), jnp.int32),
        pltpu.VMEM((K + PAD,), jnp.int32),
        pltpu.VMEM((K + PAD,), jnp.int32),
        pltpu.VMEM((K, D), jnp.float32),
        pltpu.VMEM((K, D), jnp.float32),
        pltpu.VMEM((RPW + 1, D), jnp.float32),
        pltpu.SemaphoreType.DMA,
        pltpu.SemaphoreType.DMA,
        pltpu.SemaphoreType.DMA,
    ],
)(_segmax_body)


def kernel(x, edge_index, W_gc, b_gc, W_ma, b_ma, W_ll, b_ll):
    src = edge_index[0]
    dst = edge_index[1]
    y = _matmul_bias(x, W_ma.T, b_ma.reshape(1, D))
    c_pad = _segmax(y, src, dst)
    d = _combine(x, c_pad, W_ll[:, :D].T, W_ll[:, D:].T, b_ll.reshape(1, D))
    return d


# speedup vs baseline: 9.3075x; 1.0750x over previous
import functools

import jax
import jax.numpy as jnp
from jax import lax
from jax.experimental import pallas as pl
from jax.experimental.pallas import tpu as pltpu
from jax.experimental.pallas import tpu_sc as plsc

N = 10000
E = 320000
D = 128

NC = 2
NS = 16
NW = NC * NS
RPW = 320
NPAD = NW * RPW
DUMMY = RPW

CH = 3200
NCHUNK = E // CH
NV = CH // 16
K = 256
PAD = 128
UNROLL = 8

BR = 2000


def _mm_kernel(x_ref, wt_ref, b_ref, o_ref):
    o_ref[...] = (
        jnp.dot(x_ref[...], wt_ref[...], preferred_element_type=jnp.float32)
        + b_ref[...]
    )


def _matmul_bias(x, wt, b):
    return pl.pallas_call(
        _mm_kernel,
        grid=(N // BR,),
        in_specs=[
            pl.BlockSpec((BR, D), lambda i: (i, 0)),
            pl.BlockSpec((D, D), lambda i: (0, 0)),
            pl.BlockSpec((1, D), lambda i: (0, 0)),
        ],
        out_specs=pl.BlockSpec((BR, D), lambda i: (i, 0)),
        out_shape=jax.ShapeDtypeStruct((N, D), jnp.float32),
    )(x, wt, b)


def _out_kernel(x_ref, c_ref, w1t_ref, w2t_ref, b_ref, o_ref):
    c = c_ref[...]
    c = jnp.where(c > -jnp.inf, c, 0.0)
    d = jnp.dot(x_ref[...], w1t_ref[...], preferred_element_type=jnp.float32)
    d = d + jnp.dot(c, w2t_ref[...], preferred_element_type=jnp.float32)
    d = d + b_ref[...]
    nrm = jnp.sqrt(jnp.sum(d * d, axis=1, keepdims=True))
    o_ref[...] = d / jnp.maximum(nrm, 1e-12)


def _combine(x, c_pad, w1t, w2t, b):
    return pl.pallas_call(
        _out_kernel,
        grid=(N // BR,),
        in_specs=[
            pl.BlockSpec((BR, D), lambda i: (i, 0)),
            pl.BlockSpec((BR, D), lambda i: (i, 0)),
            pl.BlockSpec((D, D), lambda i: (0, 0)),
            pl.BlockSpec((D, D), lambda i: (0, 0)),
            pl.BlockSpec((1, D), lambda i: (0, 0)),
        ],
        out_specs=pl.BlockSpec((BR, D), lambda i: (i, 0)),
        out_shape=jax.ShapeDtypeStruct((N, D), jnp.float32),
    )(x, c_pad, w1t, w2t, b)


def _segmax_body(y_hbm, src_hbm, dst_hbm, out_hbm,
                 src_buf, dst_buf, cidx0, cdst0, cidx1, cdst1,
                 rows0, rows1, acc, sem, sem_s, sem_d):
    cid = lax.axis_index("c")
    sid = lax.axis_index("s")
    wid = sid * NC + cid
    lo = wid * RPW

    neg = jnp.full((16,), -jnp.inf, dtype=jnp.float32)

    def init_acc(i, carry):
        for ch8 in range(D // 16):
            acc[i, pl.ds(ch8 * 16, 16)] = neg
        return carry

    lax.fori_loop(0, RPW + 1, init_acc, 0)

    zero16 = jnp.zeros((16,), jnp.int32)
    dummy16 = jnp.full((16,), DUMMY, jnp.int32)

    def init_ring(i, carry):
        cidx0[pl.ds(i * 16, 16)] = zero16
        cdst0[pl.ds(i * 16, 16)] = dummy16
        cidx1[pl.ds(i * 16, 16)] = zero16
        cdst1[pl.ds(i * 16, 16)] = dummy16
        return carry

    lax.fori_loop(0, (K + PAD) // 16, init_ring, 0)

    def fire(ci, rv):
        pltpu.async_copy(y_hbm.at[ci.at[pl.ds(0, K)]], rv, sem)

    def wait_fold(ci, cd, rv):
        pltpu.make_async_copy(y_hbm.at[ci.at[pl.ds(0, K)]], rv, sem).wait()

        def fold16(g, carry):
            dvec = cd[pl.ds(g * 16, 16)]
            row = [rv[g * 16, pl.ds(c * 16, 16)] for c in range(D // 16)]
            for lane in range(16):
                d_row = dvec[lane]
                cur = [acc[d_row, pl.ds(c * 16, 16)] for c in range(D // 16)]
                if lane < 15:
                    nxt = [rv[g * 16 + lane + 1, pl.ds(c * 16, 16)]
                           for c in range(D // 16)]
                for c in range(D // 16):
                    acc[d_row, pl.ds(c * 16, 16)] = jnp.maximum(cur[c], row[c])
                if lane < 15:
                    row = nxt
            return carry

        lax.fori_loop(0, K // 16, fold16, 0)

    def ov_copy(src_ci, src_cd, dst_ci, dst_cd):
        for t in range(PAD // 16):
            ov_i = src_ci[pl.ds(K + t * 16, 16)]
            ov_d = src_cd[pl.ds(K + t * 16, 16)]
            dst_ci[pl.ds(t * 16, 16)] = ov_i
            dst_cd[pl.ds(t * 16, 16)] = ov_d

    def start_fetch(c, par):
        pltpu.async_copy(src_hbm.at[pl.ds(c * CH, CH)], src_buf.at[par], sem_s)
        pltpu.async_copy(dst_hbm.at[pl.ds(c * CH, CH)], dst_buf.at[par], sem_d)

    start_fetch(0, 0)

    def chunk_body(chunk, carry):
        par = jnp.bitwise_and(chunk, 1)
        pltpu.make_async_copy(src_hbm.at[pl.ds(chunk * CH, CH)],
                              src_buf.at[par], sem_s).wait()
        pltpu.make_async_copy(dst_hbm.at[pl.ds(chunk * CH, CH)],
                              dst_buf.at[par], sem_d).wait()

        @pl.when(chunk + 1 < NCHUNK)
        def _():
            start_fetch(chunk + 1, 1 - par)

        def grp_body(g, carry):
            nacc, pend, bp = carry
            svs, dls, ms, cnts = [], [], [], []
            for u in range(UNROLL):
                i = g * UNROLL + u
                dv = dst_buf[par, pl.ds(i * 16, 16)]
                sv = src_buf[par, pl.ds(i * 16, 16)]
                dl = dv - lo
                m = dl.astype(jnp.uint32) < jnp.uint32(RPW)
                cnts.append(plsc.all_reduce_population_count(m)[0])
                svs.append(sv)
                dls.append(dl)
                ms.append(m)

            def append(ci, cd):
                n = nacc
                for u in range(UNROLL):
                    plsc.store_compressed(ci.at[pl.ds(n, 16)], svs[u],
                                          mask=ms[u])
                    plsc.store_compressed(cd.at[pl.ds(n, 16)], dls[u],
                                          mask=ms[u])
                    n = n + cnts[u]
                return n

            @pl.when(bp == 0)
            def _():
                append(cidx0, cdst0)

            @pl.when(bp == 1)
            def _():
                append(cidx1, cdst1)

            n = nacc
            for u in range(UNROLL):
                n = n + cnts[u]
            full = n >= K

            @pl.when(full)
            def _():
                @pl.when((pend == 1) & (bp == 0))
                def _():
                    wait_fold(cidx1, cdst1, rows1)

                @pl.when((pend == 1) & (bp == 1))
                def _():
                    wait_fold(cidx0, cdst0, rows0)

                @pl.when(bp == 0)
                def _():
                    fire(cidx0, rows0)
                    ov_copy(cidx0, cdst0, cidx1, cdst1)

                @pl.when(bp == 1)
                def _():
                    fire(cidx1, rows1)
                    ov_copy(cidx1, cdst1, cidx0, cdst0)

            nacc2 = jnp.where(full, n - K, n)
            pend2 = jnp.where(full, 1, pend)
            bp2 = jnp.where(full, 1 - bp, bp)
            return (nacc2, pend2, bp2)

        return lax.fori_loop(0, NV // UNROLL, grp_body, carry)

    nacc, pend, bp = lax.fori_loop(0, NCHUNK, chunk_body, (0, 0, 0))

    @pl.when((pend == 1) & (bp == 0))
    def _():
        wait_fold(cidx1, cdst1, rows1)

    @pl.when((pend == 1) & (bp == 1))
    def _():
        wait_fold(cidx0, cdst0, rows0)

    @pl.when(bp == 0)
    def _():
        fire(cidx0, rows0)
        wait_fold(cidx0, cdst0, rows0)

    @pl.when(bp == 1)
    def _():
        fire(cidx1, rows1)
        wait_fold(cidx1, cdst1, rows1)

    pltpu.sync_copy(acc.at[pl.ds(0, RPW)], out_hbm.at[pl.ds(lo, RPW)])


_segmax = functools.partial(
    pl.kernel,
    out_type=jax.ShapeDtypeStruct((NPAD, D), jnp.float32),
    mesh=plsc.VectorSubcoreMesh(core_axis_name="c", subcore_axis_name="s"),
    compiler_params=pltpu.CompilerParams(needs_layout_passes=False),
    scratch_types=[
        pltpu.VMEM((2, CH), jnp.int32),
        pltpu.VMEM((2, CH), jnp.int32),
        pltpu.VMEM((K + PAD,), jnp.int32),
        pltpu.VMEM((K + PAD,), jnp.int32),
        pltpu.VMEM((K + PAD,), jnp.int32),
        pltpu.VMEM((K + PAD,), jnp.int32),
        pltpu.VMEM((K, D), jnp.float32),
        pltpu.VMEM((K, D), jnp.float32),
        pltpu.VMEM((RPW + 1, D), jnp.float32),
        pltpu.SemaphoreType.DMA,
        pltpu.SemaphoreType.DMA,
        pltpu.SemaphoreType.DMA,
    ],
)(_segmax_body)


def kernel(x, edge_index, W_gc, b_gc, W_ma, b_ma, W_ll, b_ll):
    src = edge_index[0]
    dst = edge_index[1]
    y = _matmul_bias(x, W_ma.T, b_ma.reshape(1, D))
    c_pad = _segmax(y, src, dst)
    d = _combine(x, c_pad, W_ll[:, :D].T, W_ll[:, D:].T, b_ll.reshape(1, D))
    return d
